# Initial kernel scaffold; baseline (speedup 1.0000x reference)
import jax
import jax.numpy as jnp
from jax.experimental import pallas as pl


def kernel(x_nchw, conv1_w, bn1_gamma, bn1_beta, bn1_mean, bn1_var):
    raise NotImplementedError("write your pallas kernel here")


# speedup vs baseline: 1.7472x; 1.7472x over previous
import jax
import jax.numpy as jnp
from jax import lax
from jax.experimental import pallas as pl
from jax.experimental.pallas import tpu as pltpu

_K = 7
_S = 2
_P = 3
_COUT = 64
_EPS = 1e-5


def _stem_kernel(p_ref, w_ref, b_ref, o_ref):
    ph, pw = o_ref.shape[1], o_ref.shape[2]
    acc = jnp.dot(p_ref[...], w_ref[...], preferred_element_type=jnp.float32)
    y = jnp.maximum(acc + b_ref[...], 0.0).astype(jnp.bfloat16)
    yr = y.reshape(ph, 2, pw, 2 * _COUT)
    e = yr[:, 0]
    o = yr[:, 1]
    o_up = jnp.concatenate([o[:1], o[:-1]], axis=0)
    r = jnp.maximum(jnp.maximum(e, o), o_up)
    a = r[:, :, :_COUT]
    b = r[:, :, _COUT:]
    b_left = jnp.concatenate([b[:, :1], b[:, :-1]], axis=1)
    o_ref[0] = jnp.maximum(jnp.maximum(a, b), b_left)


def kernel(x_nchw, conv1_w, bn1_gamma, bn1_beta, bn1_mean, bn1_var):
    N, C, H, W = x_nchw.shape
    OH = (H + 2 * _P - _K) // _S + 1
    OW = (W + 2 * _P - _K) // _S + 1
    assert OH % 2 == 0 and OW % 2 == 0, "stem expects even conv output dims"
    PH, PW = OH // 2, OW // 2
    KT = _K * _K * C

    x = jnp.transpose(x_nchw.astype(jnp.bfloat16), (0, 2, 3, 1))
    patches = lax.conv_general_dilated_patches(
        x, filter_shape=(_K, _K), window_strides=(_S, _S),
        padding=((_P, _P), (_P, _P)),
        dimension_numbers=("NHWC", "HWIO", "NHWC"))
    patches = patches.reshape(N * OH * PW, 2 * KT)

    scale = bn1_gamma / jnp.sqrt(bn1_var + _EPS)
    w2d = (conv1_w.transpose(2, 0, 1, 3).reshape(KT, _COUT)
           * scale[None, :]).astype(jnp.bfloat16)
    w_pair = jnp.zeros((2 * KT, 2 * _COUT), jnp.bfloat16)
    w_pair = w_pair.at[:KT, :_COUT].set(w2d).at[KT:, _COUT:].set(w2d)
    bias = bn1_beta - bn1_mean * scale
    b2 = jnp.tile(bias, 2).reshape(1, 2 * _COUT).astype(jnp.float32)

    pooled = pl.pallas_call(
        _stem_kernel,
        out_shape=jax.ShapeDtypeStruct((N, PH, PW, _COUT), jnp.bfloat16),
        grid=(N,),
        in_specs=[
            pl.BlockSpec((OH * PW, 2 * KT), lambda i: (i, 0)),
            pl.BlockSpec((2 * KT, 2 * _COUT), lambda i: (0, 0)),
            pl.BlockSpec((1, 2 * _COUT), lambda i: (0, 0)),
        ],
        out_specs=pl.BlockSpec((1, PH, PW, _COUT), lambda i: (i, 0, 0, 0)),
        compiler_params=pltpu.CompilerParams(
            dimension_semantics=("parallel",),
            allow_input_fusion=[True, False, False],
        ),
    )(patches, w_pair, b2)

    return jnp.transpose(pooled.astype(jnp.float32), (0, 3, 1, 2))


# speedup vs baseline: 14.2028x; 8.1289x over previous
import jax
import jax.numpy as jnp
from jax import lax
from jax.experimental import pallas as pl
from jax.experimental.pallas import tpu as pltpu

_K = 7
_S = 2
_P = 3
_COUT = 64
_EPS = 1e-5
_LANE = 128


def _stem_kernel(x_ref, w_ref, b_ref, sel_ref, pse_ref, o_ref):
    _, C, H, W = x_ref.shape
    OH = (H + 2 * _P - _K) // _S + 1
    OW = (W + 2 * _P - _K) // _S + 1
    PH, PW = OH // 2, OW // 2
    HP = H + 2 * _P

    xb = x_ref[0].astype(jnp.bfloat16)
    xp = jnp.pad(xb, ((0, 0), (_P, _P), (_P, _P)))
    xr = xp.reshape(C, HP // 2, 2, W + 2 * _P)
    xe = xr[:, :, 0].reshape(C * (HP // 2), W + 2 * _P)
    xo = xr[:, :, 1].reshape(C * (HP // 2), W + 2 * _P)
    ge = jnp.dot(xe, sel_ref[...],
                 preferred_element_type=jnp.float32).astype(jnp.bfloat16)
    go = jnp.dot(xo, sel_ref[...],
                 preferred_element_type=jnp.float32).astype(jnp.bfloat16)

    taps = []
    for c in range(C):
        for ky in range(_K):
            src = go if ky % 2 else ge
            j = ky // 2
            rows = src[c * (HP // 2) + j: c * (HP // 2) + j + OH]
            for kx in range(_K):
                taps.append(rows[:, kx * _LANE: kx * _LANE + OW])
    patches = jnp.stack(taps, axis=0)

    y = lax.dot_general(w_ref[...], patches,
                        dimension_numbers=(((1,), (0,)), ((), ())),
                        preferred_element_type=jnp.float32)
    y = jnp.maximum(y + b_ref[...][:, :, None], 0.0).astype(jnp.bfloat16)

    yr = y.reshape(_COUT, PH, 2, OW)
    ye = yr[:, :, 0]
    yo = yr[:, :, 1]
    yu = jnp.concatenate([yo[:, :1], yo[:, :-1]], axis=1)
    r = jnp.maximum(jnp.maximum(ye, yo), yu)
    rb = jnp.dot(r.reshape(_COUT * PH, OW), pse_ref[...],
                 preferred_element_type=jnp.float32
                 ).astype(jnp.bfloat16)
    a = rb[:, :PW]
    b = rb[:, _LANE: _LANE + PW]
    bl = jnp.concatenate([b[:, :1], b[:, :-1]], axis=1)
    out = jnp.maximum(jnp.maximum(a, b), bl)
    o_ref[0] = out.reshape(_COUT, PH, PW).astype(jnp.float32)


def kernel(x_nchw, conv1_w, bn1_gamma, bn1_beta, bn1_mean, bn1_var):
    N, C, H, W = x_nchw.shape
    OH = (H + 2 * _P - _K) // _S + 1
    OW = (W + 2 * _P - _K) // _S + 1
    assert OH % 2 == 0 and OW % 2 == 0, "stem expects even conv output dims"
    PH, PW = OH // 2, OW // 2
    KT = _K * _K * C

    scale = bn1_gamma / jnp.sqrt(bn1_var + _EPS)
    wmat = (conv1_w.transpose(3, 2, 0, 1).reshape(_COUT, KT)
            * scale[:, None]).astype(jnp.bfloat16)
    bias = (bn1_beta - bn1_mean * scale).reshape(_COUT, 1).astype(jnp.float32)

    wp = W + 2 * _P
    col = jnp.arange(_K * _LANE)
    ox, kx = col % _LANE, col // _LANE
    sel = ((jnp.arange(wp)[:, None] == _S * ox[None, :] + kx[None, :])
           & (ox[None, :] < OW)).astype(jnp.bfloat16)
    pcol = jnp.arange(2 * _LANE)
    px, par = pcol % _LANE, pcol // _LANE
    psel = ((jnp.arange(OW)[:, None] == _S * px[None, :] + par[None, :])
            & (px[None, :] < PW)).astype(jnp.bfloat16)

    return pl.pallas_call(
        _stem_kernel,
        out_shape=jax.ShapeDtypeStruct((N, _COUT, PH, PW), jnp.float32),
        grid=(N,),
        in_specs=[
            pl.BlockSpec((1, C, H, W), lambda i: (i, 0, 0, 0)),
            pl.BlockSpec((_COUT, KT), lambda i: (0, 0)),
            pl.BlockSpec((_COUT, 1), lambda i: (0, 0)),
            pl.BlockSpec((wp, _K * _LANE), lambda i: (0, 0)),
            pl.BlockSpec((OW, 2 * _LANE), lambda i: (0, 0)),
        ],
        out_specs=pl.BlockSpec((1, _COUT, PH, PW), lambda i: (i, 0, 0, 0)),
        compiler_params=pltpu.CompilerParams(
            dimension_semantics=("parallel",),
        ),
    )(x_nchw, wmat, bias, sel, psel)


# speedup vs baseline: 19.8948x; 1.4008x over previous
import jax
import jax.numpy as jnp
from jax import lax
from jax.experimental import pallas as pl
from jax.experimental.pallas import tpu as pltpu

_K = 7
_S = 2
_P = 3
_COUT = 64
_EPS = 1e-5
_LANE = 128
_GSLOT = 32


def _stem_kernel(x_ref, w_ref, b_ref, srow_ref, sel_ref, pse_ref, o_ref):
    _, C, H, W = x_ref.shape
    OH = (H + 2 * _P - _K) // _S + 1
    OW = (W + 2 * _P - _K) // _S + 1
    PH, PW = OH // 2, OW // 2

    xb = x_ref[0].astype(jnp.bfloat16)
    xp = jnp.pad(xb, ((0, 0), (_P, _P), (_P, _P)))
    xp2 = xp.reshape(C * (H + 2 * _P), W + 2 * _P)
    pr = jnp.dot(srow_ref[...], xp2,
                 preferred_element_type=jnp.float32).astype(jnp.bfloat16)
    pc = jnp.dot(pr, sel_ref[...],
                 preferred_element_type=jnp.float32).astype(jnp.bfloat16)
    pc3 = pc.reshape(OH, _GSLOT, _K * _LANE)
    patches = jnp.concatenate(
        [pc3[:, :, kx * _LANE:(kx + 1) * _LANE] for kx in range(_K)], axis=1)

    y = lax.dot_general(patches, w_ref[...],
                        dimension_numbers=(((1,), (1,)), ((), ())),
                        preferred_element_type=jnp.float32)
    y = jnp.maximum(y + b_ref[...][None], 0.0).astype(jnp.bfloat16)

    yr = y.reshape(PH, 2, _LANE, _COUT)
    ye = yr[:, 0]
    yo = yr[:, 1]
    yu = jnp.concatenate([yo[:1], yo[:-1]], axis=0)
    r = jnp.maximum(jnp.maximum(ye, yo), yu)
    rp = lax.dot_general(r, pse_ref[...],
                         dimension_numbers=(((1,), (0,)), ((), ())),
                         preferred_element_type=jnp.float32
                         ).astype(jnp.bfloat16)
    a = rp[:, :, :PW]
    b = rp[:, :, _LANE:_LANE + PW]
    bl = jnp.concatenate([b[:, :, :1], b[:, :, :-1]], axis=2)
    out = jnp.maximum(jnp.maximum(a, b), bl)
    o_ref[0] = jnp.transpose(out, (1, 0, 2)).astype(jnp.float32)


def kernel(x_nchw, conv1_w, bn1_gamma, bn1_beta, bn1_mean, bn1_var):
    N, C, H, W = x_nchw.shape
    OH = (H + 2 * _P - _K) // _S + 1
    OW = (W + 2 * _P - _K) // _S + 1
    assert OH % 2 == 0 and OW % 2 == 0, "stem expects even conv output dims"
    PH, PW = OH // 2, OW // 2
    HP = H + 2 * _P

    scale = bn1_gamma / jnp.sqrt(bn1_var + _EPS)
    wfold = conv1_w * scale[None, None, None, :]
    wmat = jnp.zeros((_COUT, _K * _GSLOT), jnp.float32)
    wperm = wfold.transpose(3, 1, 2, 0).reshape(_COUT, _K, C * _K)
    for kx in range(_K):
        wmat = wmat.at[:, kx * _GSLOT: kx * _GSLOT + C * _K].set(wperm[:, kx])
    wmat = wmat.astype(jnp.bfloat16)
    bias = (bn1_beta - bn1_mean * scale).reshape(1, _COUT).astype(jnp.float32)

    gy = jnp.arange(OH * _GSLOT)
    oy, slot = gy // _GSLOT, gy % _GSLOT
    cc, ky = slot // _K, slot % _K
    srow = ((jnp.arange(C * HP)[None, :]
             == (cc * HP + _S * oy + ky)[:, None])
            & (slot[:, None] < C * _K)).astype(jnp.bfloat16)
    col = jnp.arange(_K * _LANE)
    ox, kx = col % _LANE, col // _LANE
    sel = ((jnp.arange(HP)[:, None] == _S * ox[None, :] + kx[None, :])
           & (ox[None, :] < OW)).astype(jnp.bfloat16)
    pcol = jnp.arange(2 * _LANE)
    px, par = pcol % _LANE, pcol // _LANE
    psel = ((jnp.arange(_LANE)[:, None] == _S * px[None, :] + par[None, :])
            & (px[None, :] < PW)).astype(jnp.bfloat16)

    return pl.pallas_call(
        _stem_kernel,
        out_shape=jax.ShapeDtypeStruct((N, _COUT, PH, PW), jnp.float32),
        grid=(N,),
        in_specs=[
            pl.BlockSpec((1, C, H, W), lambda i: (i, 0, 0, 0)),
            pl.BlockSpec((_COUT, _K * _GSLOT), lambda i: (0, 0)),
            pl.BlockSpec((1, _COUT), lambda i: (0, 0)),
            pl.BlockSpec((OH * _GSLOT, C * HP), lambda i: (0, 0)),
            pl.BlockSpec((HP, _K * _LANE), lambda i: (0, 0)),
            pl.BlockSpec((_LANE, 2 * _LANE), lambda i: (0, 0)),
        ],
        out_specs=pl.BlockSpec((1, _COUT, PH, PW), lambda i: (i, 0, 0, 0)),
        compiler_params=pltpu.CompilerParams(
            dimension_semantics=("parallel",),
        ),
    )(x_nchw, wmat, bias, srow, sel, psel)


# speedup vs baseline: 20.6728x; 1.0391x over previous
import jax
import jax.numpy as jnp
from jax import lax
from jax.experimental import pallas as pl
from jax.experimental.pallas import tpu as pltpu

_K = 7
_S = 2
_P = 3
_COUT = 64
_EPS = 1e-5
_LANE = 128
_GSLOT = 32


def _stem_kernel(x_ref, w_ref, b_ref, srow_ref, sel_ref, pse_ref, o_ref):
    IB, C, H, W = x_ref.shape
    OH = (H + 2 * _P - _K) // _S + 1
    OW = (W + 2 * _P - _K) // _S + 1
    PH, PW = OH // 2, OW // 2

    for img in range(IB):
        xb = x_ref[img].astype(jnp.bfloat16)
        xp = jnp.pad(xb, ((0, 0), (_P, _P), (_P, _P)))
        xp2 = xp.reshape(C * (H + 2 * _P), W + 2 * _P)
        pr = jnp.dot(srow_ref[...], xp2,
                     preferred_element_type=jnp.float32).astype(jnp.bfloat16)
        pc = jnp.dot(pr, sel_ref[...],
                     preferred_element_type=jnp.float32).astype(jnp.bfloat16)
        pc3 = pc.reshape(OH, _GSLOT, _K * _LANE)
        patches = jnp.concatenate(
            [pc3[:, :, kx * _LANE:(kx + 1) * _LANE] for kx in range(_K)],
            axis=1)

        y = lax.dot_general(patches, w_ref[...],
                            dimension_numbers=(((1,), (1,)), ((), ())),
                            preferred_element_type=jnp.float32)
        y = jnp.maximum(y + b_ref[...][None], 0.0).astype(jnp.bfloat16)

        yr = y.reshape(PH, 2, _LANE, _COUT)
        ye = yr[:, 0]
        yo = yr[:, 1]
        yu = jnp.concatenate([yo[:1], yo[:-1]], axis=0)
        r = jnp.maximum(jnp.maximum(ye, yo), yu)
        rp = lax.dot_general(r, pse_ref[...],
                             dimension_numbers=(((1,), (0,)), ((), ())),
                             preferred_element_type=jnp.float32
                             ).astype(jnp.bfloat16)
        a = rp[:, :, :PW]
        b = rp[:, :, _LANE:_LANE + PW]
        bl = jnp.concatenate([b[:, :, :1], b[:, :, :-1]], axis=2)
        out = jnp.maximum(jnp.maximum(a, b), bl)
        o_ref[img] = jnp.transpose(out, (1, 0, 2)).astype(jnp.float32)


def kernel(x_nchw, conv1_w, bn1_gamma, bn1_beta, bn1_mean, bn1_var):
    N, C, H, W = x_nchw.shape
    OH = (H + 2 * _P - _K) // _S + 1
    OW = (W + 2 * _P - _K) // _S + 1
    assert OH % 2 == 0 and OW % 2 == 0, "stem expects even conv output dims"
    PH, PW = OH // 2, OW // 2
    HP = H + 2 * _P

    scale = bn1_gamma / jnp.sqrt(bn1_var + _EPS)
    wfold = conv1_w * scale[None, None, None, :]
    wmat = jnp.zeros((_COUT, _K * _GSLOT), jnp.float32)
    wperm = wfold.transpose(3, 1, 2, 0).reshape(_COUT, _K, C * _K)
    for kx in range(_K):
        wmat = wmat.at[:, kx * _GSLOT: kx * _GSLOT + C * _K].set(wperm[:, kx])
    wmat = wmat.astype(jnp.bfloat16)
    bias = (bn1_beta - bn1_mean * scale).reshape(1, _COUT).astype(jnp.float32)

    gy = jnp.arange(OH * _GSLOT)
    oy, slot = gy // _GSLOT, gy % _GSLOT
    cc, ky = slot // _K, slot % _K
    srow = ((jnp.arange(C * HP)[None, :]
             == (cc * HP + _S * oy + ky)[:, None])
            & (slot[:, None] < C * _K)).astype(jnp.bfloat16)
    col = jnp.arange(_K * _LANE)
    ox, kx = col % _LANE, col // _LANE
    sel = ((jnp.arange(HP)[:, None] == _S * ox[None, :] + kx[None, :])
           & (ox[None, :] < OW)).astype(jnp.bfloat16)
    pcol = jnp.arange(2 * _LANE)
    px, par = pcol % _LANE, pcol // _LANE
    psel = ((jnp.arange(_LANE)[:, None] == _S * px[None, :] + par[None, :])
            & (px[None, :] < PW)).astype(jnp.bfloat16)

    ib = 4 if N % 4 == 0 else 1
    return pl.pallas_call(
        _stem_kernel,
        out_shape=jax.ShapeDtypeStruct((N, _COUT, PH, PW), jnp.float32),
        grid=(N // ib,),
        in_specs=[
            pl.BlockSpec((ib, C, H, W), lambda i: (i, 0, 0, 0)),
            pl.BlockSpec((_COUT, _K * _GSLOT), lambda i: (0, 0)),
            pl.BlockSpec((1, _COUT), lambda i: (0, 0)),
            pl.BlockSpec((OH * _GSLOT, C * HP), lambda i: (0, 0)),
            pl.BlockSpec((HP, _K * _LANE), lambda i: (0, 0)),
            pl.BlockSpec((_LANE, 2 * _LANE), lambda i: (0, 0)),
        ],
        out_specs=pl.BlockSpec((ib, _COUT, PH, PW), lambda i: (i, 0, 0, 0)),
        compiler_params=pltpu.CompilerParams(
            dimension_semantics=("parallel",),
        ),
    )(x_nchw, wmat, bias, srow, sel, psel)


# speedup vs baseline: 24.8762x; 1.2033x over previous
import jax
import jax.numpy as jnp
from jax import lax
from jax.experimental import pallas as pl
from jax.experimental.pallas import tpu as pltpu

_K = 7
_S = 2
_P = 3
_COUT = 64
_EPS = 1e-5
_LANE = 128
_KSLOT = 8


def _stem_kernel(x_ref, w_ref, b_ref, srow_ref, sel_ref, pse_ref, o_ref):
    IB, C, H, W = x_ref.shape
    OH = (H + 2 * _P - _K) // _S + 1
    OW = (W + 2 * _P - _K) // _S + 1
    PH, PW = OH // 2, OW // 2

    for img in range(IB):
        xb = x_ref[img].astype(jnp.bfloat16)
        xp = jnp.pad(xb, ((0, 0), (_P, _P), (_P, _P)))
        pieces = []
        for c in range(C):
            pr = jnp.dot(srow_ref[...], xp[c],
                         preferred_element_type=jnp.float32
                         ).astype(jnp.bfloat16)
            pc = jnp.dot(pr, sel_ref[...],
                         preferred_element_type=jnp.float32)
            pc3 = pc.reshape(OH, _KSLOT, _K * _LANE)
            for kx in range(_K):
                pieces.append(pc3[:, :, kx * _LANE:(kx + 1) * _LANE])
        patches = jnp.concatenate(pieces, axis=1).astype(jnp.bfloat16)

        y = lax.dot_general(patches, w_ref[...],
                            dimension_numbers=(((1,), (1,)), ((), ())),
                            preferred_element_type=jnp.float32)
        y = jnp.maximum(y + b_ref[...][None], 0.0).astype(jnp.bfloat16)

        yr = y.reshape(PH, 2, _LANE, _COUT)
        ye = yr[:, 0]
        yo = yr[:, 1]
        yu = jnp.concatenate([yo[:1], yo[:-1]], axis=0)
        r = jnp.maximum(jnp.maximum(ye, yo), yu)
        rp = lax.dot_general(r, pse_ref[...],
                             dimension_numbers=(((1,), (0,)), ((), ())),
                             preferred_element_type=jnp.float32
                             ).astype(jnp.bfloat16)
        a = rp[:, :, :PW]
        b = rp[:, :, _LANE:_LANE + PW]
        bl = jnp.concatenate([b[:, :, :1], b[:, :, :-1]], axis=2)
        out = jnp.maximum(jnp.maximum(a, b), bl)
        o_ref[img] = jnp.transpose(out, (1, 0, 2)).astype(jnp.float32)


def kernel(x_nchw, conv1_w, bn1_gamma, bn1_beta, bn1_mean, bn1_var):
    N, C, H, W = x_nchw.shape
    OH = (H + 2 * _P - _K) // _S + 1
    OW = (W + 2 * _P - _K) // _S + 1
    assert OH % 2 == 0 and OW % 2 == 0, "stem expects even conv output dims"
    PH, PW = OH // 2, OW // 2
    HP = H + 2 * _P

    scale = bn1_gamma / jnp.sqrt(bn1_var + _EPS)
    wfold = (conv1_w * scale[None, None, None, :]
             ).transpose(3, 2, 1, 0)
    wmat = jnp.pad(wfold, ((0, 0), (0, 0), (0, 0), (0, _KSLOT - _K))
                   ).reshape(_COUT, C * _K * _KSLOT).astype(jnp.bfloat16)
    bias = (bn1_beta - bn1_mean * scale).reshape(1, _COUT).astype(jnp.float32)

    gy = jnp.arange(OH * _KSLOT)
    oy, ky = gy // _KSLOT, gy % _KSLOT
    srow = ((jnp.arange(HP)[None, :] == (_S * oy + ky)[:, None])
            & (ky[:, None] < _K)).astype(jnp.bfloat16)
    col = jnp.arange(_K * _LANE)
    ox, kx = col % _LANE, col // _LANE
    sel = ((jnp.arange(HP)[:, None] == _S * ox[None, :] + kx[None, :])
           & (ox[None, :] < OW)).astype(jnp.bfloat16)
    pcol = jnp.arange(2 * _LANE)
    px, par = pcol % _LANE, pcol // _LANE
    psel = ((jnp.arange(_LANE)[:, None] == _S * px[None, :] + par[None, :])
            & (px[None, :] < PW)).astype(jnp.bfloat16)

    ib = 4 if N % 4 == 0 else 1
    return pl.pallas_call(
        _stem_kernel,
        out_shape=jax.ShapeDtypeStruct((N, _COUT, PH, PW), jnp.float32),
        grid=(N // ib,),
        in_specs=[
            pl.BlockSpec((ib, C, H, W), lambda i: (i, 0, 0, 0)),
            pl.BlockSpec((_COUT, C * _K * _KSLOT), lambda i: (0, 0)),
            pl.BlockSpec((1, _COUT), lambda i: (0, 0)),
            pl.BlockSpec((OH * _KSLOT, HP), lambda i: (0, 0)),
            pl.BlockSpec((HP, _K * _LANE), lambda i: (0, 0)),
            pl.BlockSpec((_LANE, 2 * _LANE), lambda i: (0, 0)),
        ],
        out_specs=pl.BlockSpec((ib, _COUT, PH, PW), lambda i: (i, 0, 0, 0)),
        compiler_params=pltpu.CompilerParams(
            dimension_semantics=("arbitrary",),
        ),
    )(x_nchw, wmat, bias, srow, sel, psel)
